# Initial kernel scaffold; baseline (speedup 1.0000x reference)
import jax
import jax.numpy as jnp
from jax.experimental import pallas as pl


def kernel(params, scope_tokens, scope_sort, scope_padding_mask, scope_reference_mask, hole_tokens, hole_padding_mask, hole_reference_mask):
    raise NotImplementedError("write your pallas kernel here")


# speedup vs baseline: 7.4547x; 7.4547x over previous
import jax
import jax.numpy as jnp
import numpy as np
from jax.experimental import pallas as pl
from jax.experimental.pallas import tpu as pltpu

NUM_LAYERS = 4
NUM_HEADS = 16
DIM = 1024
HEAD_DIM = 64
HALF = HEAD_DIM // 2
FFN = 4 * DIM
N_TREES = 64
SEQ = 256
B_HOLE = 16
NSTEP = N_TREES + B_HOLE


def _encoder_body(feat_ref, oh_ref, pb_ref, cos_ref, sin_ref,
                  wq_ref, wk_ref, wv_ref, wo_ref, w1_ref, w2_ref,
                  b1_ref, b2_ref, l1s_ref, l1b_ref, l2s_ref, l2b_ref,
                  scope_ref, hole_ref, x_scr):
    g = pl.program_id(0)
    l = pl.program_id(1)

    @pl.when((g == 0) & (l == 0))
    def _():
        scope_ref[...] = jnp.zeros_like(scope_ref)

    @pl.when(l == 0)
    def _():
        oh = oh_ref[0]
        storage = scope_ref[...].astype(jnp.bfloat16)
        gat = jax.lax.dot_general(oh, storage, (((1,), (0,)), ((), ())),
                                  preferred_element_type=jnp.float32)
        x_scr[...] = feat_ref[0] + gat

    x = x_scr[...]
    pb = pb_ref[0]
    cosf = cos_ref[...]
    sinf = sin_ref[...]
    lane = jax.lax.broadcasted_iota(jnp.int32, (SEQ, DIM), 1)
    lower = (lane % HEAD_DIM) < HALF

    def rot(t):
        tm = jnp.concatenate([t[:, HALF:], t[:, :HALF]], axis=1)
        tp = jnp.concatenate([t[:, -HALF:], t[:, :-HALF]], axis=1)
        sw = jnp.where(lower, tm, tp)
        return t * cosf + sw * sinf

    def ln(t, s_, b_):
        m = jnp.mean(t, axis=-1, keepdims=True)
        v = jnp.mean((t - m) ** 2, axis=-1, keepdims=True)
        return (t - m) * jax.lax.rsqrt(v + 1e-5) * s_ + b_

    def mm(a, b):
        return jax.lax.dot_general(a.astype(jnp.bfloat16), b,
                                   (((1,), (0,)), ((), ())),
                                   preferred_element_type=jnp.float32)

    h = ln(x, l1s_ref[0], l1b_ref[0])
    q = rot(mm(h, wq_ref[0]))
    k = rot(mm(h, wk_ref[0]))
    v = mm(h, wv_ref[0])
    qb = (q * (1.0 / np.sqrt(HEAD_DIM))).astype(jnp.bfloat16)
    kb = k.astype(jnp.bfloat16)
    vb = v.astype(jnp.bfloat16)
    outs = []
    for hd in range(NUM_HEADS):
        sl = slice(hd * HEAD_DIM, (hd + 1) * HEAD_DIM)
        sc = jax.lax.dot_general(qb[:, sl], kb[:, sl],
                                 (((1,), (1,)), ((), ())),
                                 preferred_element_type=jnp.float32)
        sc = sc + pb
        sc = sc - jnp.max(sc, axis=-1, keepdims=True)
        e = jnp.exp(sc)
        a = e / jnp.sum(e, axis=-1, keepdims=True)
        outs.append(jax.lax.dot_general(a.astype(jnp.bfloat16), vb[:, sl],
                                        (((1,), (0,)), ((), ())),
                                        preferred_element_type=jnp.float32))
    o = jnp.concatenate(outs, axis=1)
    x = x + mm(o, wo_ref[0])
    h2 = ln(x, l2s_ref[0], l2b_ref[0])
    t1 = mm(h2, w1_ref[0]) + b1_ref[0]
    x = x + mm(jax.nn.gelu(t1), w2_ref[0]) + b2_ref[0]
    x_scr[...] = x

    @pl.when((l == NUM_LAYERS - 1) & (g < N_TREES))
    def _():
        scope_ref[pl.ds(g, 1), :] = x[0:1, :]

    @pl.when((l == NUM_LAYERS - 1) & (g >= N_TREES))
    def _():
        hole_ref[pl.ds(g - N_TREES, 1), :] = x[0:1, :]


def _full(shape):
    n = len(shape)
    return pl.BlockSpec(shape, lambda g, l: (0,) * n)


def _per_layer(shape):
    return pl.BlockSpec((1,) + shape, lambda g, l: (l, 0, 0))


def kernel(params, scope_tokens, scope_sort, scope_padding_mask,
           scope_reference_mask, hole_tokens, hole_padding_mask,
           hole_reference_mask):
    emb = params['emb']
    layers = params['layers']
    order = jnp.argsort(scope_sort)
    inv_order = jnp.argsort(order)

    scope_feat = jnp.take(emb, scope_tokens, axis=0).sum(axis=2)
    hole_feat = jnp.take(emb, hole_tokens, axis=0).sum(axis=2)
    scope_feat = jnp.where(scope_reference_mask[..., None], 0.0, scope_feat)
    hole_feat = jnp.where(hole_reference_mask[..., None], 0.0, hole_feat)

    ids_scope = scope_tokens[:, :, 1]
    valid_scope = scope_reference_mask & (scope_sort[ids_scope] < scope_sort[:, None])
    ids_hole = hole_tokens[:, :, 1]

    feat_all = jnp.concatenate([jnp.take(scope_feat, order, axis=0), hole_feat], 0)
    ids_all = jnp.concatenate([jnp.take(inv_order[ids_scope], order, axis=0),
                               inv_order[ids_hole]], 0)
    valid_all = jnp.concatenate([jnp.take(valid_scope, order, axis=0),
                                 hole_reference_mask], 0)
    onehot = ((ids_all[..., None] == jnp.arange(N_TREES)[None, None, :])
              & valid_all[..., None]).astype(jnp.bfloat16)

    pad_all = jnp.concatenate([jnp.take(scope_padding_mask, order, axis=0),
                               hole_padding_mask], 0)
    pbias = jnp.where(pad_all, 0.0, -1e9).astype(jnp.float32).reshape(NSTEP, 1, SEQ)

    inv = 1.0 / (10000.0 ** (jnp.arange(0, HEAD_DIM, 2, dtype=jnp.float32) / HEAD_DIM))
    f = jnp.arange(SEQ, dtype=jnp.float32)[:, None] * inv[None, :]
    cos, sin = jnp.cos(f), jnp.sin(f)
    cosf = jnp.tile(jnp.concatenate([cos, cos], 1), (1, NUM_HEADS))
    sinf = jnp.tile(jnp.concatenate([-sin, sin], 1), (1, NUM_HEADS))

    j = np.arange(HEAD_DIM)
    perm_in_head = np.where(j < HALF, 2 * j, 2 * (j - HALF) + 1)
    permcols = (np.arange(NUM_HEADS)[:, None] * HEAD_DIM
                + perm_in_head[None, :]).reshape(-1)

    wq = jnp.stack([p['Wq'][:, permcols] for p in layers]).astype(jnp.bfloat16)
    wk = jnp.stack([p['Wk'][:, permcols] for p in layers]).astype(jnp.bfloat16)
    wv = jnp.stack([p['Wv'] for p in layers]).astype(jnp.bfloat16)
    wo = jnp.stack([p['Wo'] for p in layers]).astype(jnp.bfloat16)
    w1 = jnp.stack([p['W1'] for p in layers]).astype(jnp.bfloat16)
    w2 = jnp.stack([p['W2'] for p in layers]).astype(jnp.bfloat16)
    b1 = jnp.stack([p['b1'] for p in layers]).reshape(NUM_LAYERS, 1, FFN)
    b2 = jnp.stack([p['b2'] for p in layers]).reshape(NUM_LAYERS, 1, DIM)
    l1s = jnp.stack([p['ln1_s'] for p in layers]).reshape(NUM_LAYERS, 1, DIM)
    l1b = jnp.stack([p['ln1_b'] for p in layers]).reshape(NUM_LAYERS, 1, DIM)
    l2s = jnp.stack([p['ln2_s'] for p in layers]).reshape(NUM_LAYERS, 1, DIM)
    l2b = jnp.stack([p['ln2_b'] for p in layers]).reshape(NUM_LAYERS, 1, DIM)

    scope_sorted, hole_reprs = pl.pallas_call(
        _encoder_body,
        grid=(NSTEP, NUM_LAYERS),
        in_specs=[
            pl.BlockSpec((1, SEQ, DIM), lambda g, l: (g, 0, 0)),
            pl.BlockSpec((1, SEQ, N_TREES), lambda g, l: (g, 0, 0)),
            pl.BlockSpec((1, 1, SEQ), lambda g, l: (g, 0, 0)),
            _full((SEQ, DIM)),
            _full((SEQ, DIM)),
            _per_layer((DIM, DIM)),
            _per_layer((DIM, DIM)),
            _per_layer((DIM, DIM)),
            _per_layer((DIM, DIM)),
            _per_layer((DIM, FFN)),
            _per_layer((FFN, DIM)),
            _per_layer((1, FFN)),
            _per_layer((1, DIM)),
            _per_layer((1, DIM)),
            _per_layer((1, DIM)),
            _per_layer((1, DIM)),
            _per_layer((1, DIM)),
        ],
        out_specs=[
            _full((N_TREES, DIM)),
            _full((B_HOLE, DIM)),
        ],
        out_shape=[
            jax.ShapeDtypeStruct((N_TREES, DIM), jnp.float32),
            jax.ShapeDtypeStruct((B_HOLE, DIM), jnp.float32),
        ],
        scratch_shapes=[pltpu.VMEM((SEQ, DIM), jnp.float32)],
        compiler_params=pltpu.CompilerParams(
            dimension_semantics=("arbitrary", "arbitrary")),
    )(feat_all, onehot, pbias, cosf, sinf, wq, wk, wv, wo, w1, w2,
      b1, b2, l1s, l1b, l2s, l2b)

    scope_reprs = jnp.take(scope_sorted, inv_order, axis=0)
    return scope_reprs, hole_reprs


# speedup vs baseline: 8.1628x; 1.0950x over previous
import functools

import jax
import jax.numpy as jnp
import numpy as np
from jax.experimental import pallas as pl
from jax.experimental.pallas import tpu as pltpu
from jax.experimental.pallas import tpu_sc as plsc

NUM_LAYERS = 4
NUM_HEADS = 16
DIM = 1024
HEAD_DIM = 64
HALF = HEAD_DIM // 2
FFN = 4 * DIM
N_TREES = 64
SEQ = 256
B_HOLE = 16
NSTEP = N_TREES + B_HOLE


def _encoder_body(feat_ref, oh_ref, pb_ref, cos_ref, sin_ref,
                  wq_ref, wk_ref, wv_ref, wo_ref, w1_ref, w2_ref,
                  b1_ref, b2_ref, l1s_ref, l1b_ref, l2s_ref, l2b_ref,
                  scope_ref, hole_ref, x_scr):
    g = pl.program_id(0)
    l = pl.program_id(1)

    @pl.when((g == 0) & (l == 0))
    def _():
        scope_ref[...] = jnp.zeros_like(scope_ref)

    @pl.when(l == 0)
    def _():
        oh = oh_ref[0]
        storage = scope_ref[...].astype(jnp.bfloat16)
        gat = jax.lax.dot_general(oh, storage, (((1,), (0,)), ((), ())),
                                  preferred_element_type=jnp.float32)
        x_scr[...] = feat_ref[0] + gat

    x = x_scr[...]
    pb = pb_ref[0]
    cosf = cos_ref[...]
    sinf = sin_ref[...]
    lane = jax.lax.broadcasted_iota(jnp.int32, (SEQ, DIM), 1)
    lower = (lane % HEAD_DIM) < HALF

    def rot(t):
        tm = jnp.concatenate([t[:, HALF:], t[:, :HALF]], axis=1)
        tp = jnp.concatenate([t[:, -HALF:], t[:, :-HALF]], axis=1)
        sw = jnp.where(lower, tm, tp)
        return t * cosf + sw * sinf

    def ln(t, s_, b_):
        m = jnp.mean(t, axis=-1, keepdims=True)
        v = jnp.mean((t - m) ** 2, axis=-1, keepdims=True)
        return (t - m) * jax.lax.rsqrt(v + 1e-5) * s_ + b_

    def mm(a, b):
        return jax.lax.dot_general(a.astype(jnp.bfloat16), b,
                                   (((1,), (0,)), ((), ())),
                                   preferred_element_type=jnp.float32)

    h = ln(x, l1s_ref[0], l1b_ref[0])
    q = rot(mm(h, wq_ref[0]))
    k = rot(mm(h, wk_ref[0]))
    v = mm(h, wv_ref[0])
    qb = (q * (1.0 / np.sqrt(HEAD_DIM))).astype(jnp.bfloat16)
    kb = k.astype(jnp.bfloat16)
    vb = v.astype(jnp.bfloat16)
    outs = []
    for hd in range(NUM_HEADS):
        sl = slice(hd * HEAD_DIM, (hd + 1) * HEAD_DIM)
        sc = jax.lax.dot_general(qb[:, sl], kb[:, sl],
                                 (((1,), (1,)), ((), ())),
                                 preferred_element_type=jnp.float32)
        sc = sc + pb
        sc = sc - jnp.max(sc, axis=-1, keepdims=True)
        e = jnp.exp(sc)
        a = e / jnp.sum(e, axis=-1, keepdims=True)
        outs.append(jax.lax.dot_general(a.astype(jnp.bfloat16), vb[:, sl],
                                        (((1,), (0,)), ((), ())),
                                        preferred_element_type=jnp.float32))
    o = jnp.concatenate(outs, axis=1)
    x = x + mm(o, wo_ref[0])
    h2 = ln(x, l2s_ref[0], l2b_ref[0])
    t1 = mm(h2, w1_ref[0]) + b1_ref[0]
    x = x + mm(jax.nn.gelu(t1), w2_ref[0]) + b2_ref[0]
    x_scr[...] = x

    @pl.when((l == NUM_LAYERS - 1) & (g < N_TREES))
    def _():
        scope_ref[pl.ds(g, 1), :] = x[0:1, :]

    @pl.when((l == NUM_LAYERS - 1) & (g >= N_TREES))
    def _():
        hole_ref[pl.ds(g - N_TREES, 1), :] = x[0:1, :]


N_ROWS = (N_TREES + B_HOLE) * SEQ
N_WORKERS = 32
ROWS_PER_W = N_ROWS // N_WORKERS
CH = 32
VREGS_PER_CH = CH * (DIM // 16)
UNROLL = 8


def _sc_embed_body(emb_hbm, ide_hbm, ido_hbm, out_hbm, idx_v, a_v, b_v, o_v, sem):
    wid = jax.lax.axis_index("s") * 2 + jax.lax.axis_index("c")
    base_w = wid * ROWS_PER_W

    def chunk(c, carry):
        base = base_w + c * CH
        pltpu.sync_copy(ide_hbm.at[pl.ds(base, CH)], idx_v)
        pltpu.async_copy(emb_hbm.at[idx_v], a_v, sem).wait()
        pltpu.sync_copy(ido_hbm.at[pl.ds(base, CH)], idx_v)
        pltpu.async_copy(emb_hbm.at[idx_v], b_v, sem).wait()

        def add_u(i, carry2):
            for u in range(UNROLL):
                t = i * UNROLL + u
                r = t // (DIM // 16)
                col = (t % (DIM // 16)) * 16
                o_v[r, pl.ds(col, 16)] = (a_v[r, pl.ds(col, 16)]
                                          + b_v[r, pl.ds(col, 16)])
            return carry2

        jax.lax.fori_loop(0, VREGS_PER_CH // UNROLL, add_u, 0)
        pltpu.sync_copy(o_v, out_hbm.at[pl.ds(base, CH)])
        return carry

    jax.lax.fori_loop(0, ROWS_PER_W // CH, chunk, 0)


def _sc_embed(emb, ide, ido):
    kfn = functools.partial(
        pl.kernel,
        out_type=jax.ShapeDtypeStruct((N_ROWS, DIM), jnp.float32),
        mesh=plsc.VectorSubcoreMesh(core_axis_name="c", subcore_axis_name="s"),
        scratch_types=[
            pltpu.VMEM((CH,), jnp.int32),
            pltpu.VMEM((CH, DIM), jnp.float32),
            pltpu.VMEM((CH, DIM), jnp.float32),
            pltpu.VMEM((CH, DIM), jnp.float32),
            pltpu.SemaphoreType.DMA,
        ],
    )(_sc_embed_body)
    return kfn(emb, ide, ido)


def _full(shape):
    n = len(shape)
    return pl.BlockSpec(shape, lambda g, l: (0,) * n)


def _per_layer(shape):
    return pl.BlockSpec((1,) + shape, lambda g, l: (l, 0, 0))


def kernel(params, scope_tokens, scope_sort, scope_padding_mask,
           scope_reference_mask, hole_tokens, hole_padding_mask,
           hole_reference_mask):
    emb = params['emb']
    layers = params['layers']
    order = jnp.argsort(scope_sort)
    inv_order = jnp.argsort(order)

    tok_all = jnp.concatenate([scope_tokens.reshape(-1, 2),
                               hole_tokens.reshape(-1, 2)], axis=0)
    feat_flat = _sc_embed(emb, tok_all[:, 0].astype(jnp.int32),
                          tok_all[:, 1].astype(jnp.int32))
    scope_feat = feat_flat[:N_TREES * SEQ].reshape(N_TREES, SEQ, DIM)
    hole_feat = feat_flat[N_TREES * SEQ:].reshape(B_HOLE, SEQ, DIM)
    scope_feat = jnp.where(scope_reference_mask[..., None], 0.0, scope_feat)
    hole_feat = jnp.where(hole_reference_mask[..., None], 0.0, hole_feat)

    ids_scope = scope_tokens[:, :, 1]
    valid_scope = scope_reference_mask & (scope_sort[ids_scope] < scope_sort[:, None])
    ids_hole = hole_tokens[:, :, 1]

    feat_all = jnp.concatenate([jnp.take(scope_feat, order, axis=0), hole_feat], 0)
    ids_all = jnp.concatenate([jnp.take(inv_order[ids_scope], order, axis=0),
                               inv_order[ids_hole]], 0)
    valid_all = jnp.concatenate([jnp.take(valid_scope, order, axis=0),
                                 hole_reference_mask], 0)
    onehot = ((ids_all[..., None] == jnp.arange(N_TREES)[None, None, :])
              & valid_all[..., None]).astype(jnp.bfloat16)

    pad_all = jnp.concatenate([jnp.take(scope_padding_mask, order, axis=0),
                               hole_padding_mask], 0)
    pbias = jnp.where(pad_all, 0.0, -1e9).astype(jnp.float32).reshape(NSTEP, 1, SEQ)

    inv = 1.0 / (10000.0 ** (jnp.arange(0, HEAD_DIM, 2, dtype=jnp.float32) / HEAD_DIM))
    f = jnp.arange(SEQ, dtype=jnp.float32)[:, None] * inv[None, :]
    cos, sin = jnp.cos(f), jnp.sin(f)
    cosf = jnp.tile(jnp.concatenate([cos, cos], 1), (1, NUM_HEADS))
    sinf = jnp.tile(jnp.concatenate([-sin, sin], 1), (1, NUM_HEADS))

    j = np.arange(HEAD_DIM)
    perm_in_head = np.where(j < HALF, 2 * j, 2 * (j - HALF) + 1)
    permcols = (np.arange(NUM_HEADS)[:, None] * HEAD_DIM
                + perm_in_head[None, :]).reshape(-1)

    wq = jnp.stack([p['Wq'][:, permcols] for p in layers]).astype(jnp.bfloat16)
    wk = jnp.stack([p['Wk'][:, permcols] for p in layers]).astype(jnp.bfloat16)
    wv = jnp.stack([p['Wv'] for p in layers]).astype(jnp.bfloat16)
    wo = jnp.stack([p['Wo'] for p in layers]).astype(jnp.bfloat16)
    w1 = jnp.stack([p['W1'] for p in layers]).astype(jnp.bfloat16)
    w2 = jnp.stack([p['W2'] for p in layers]).astype(jnp.bfloat16)
    b1 = jnp.stack([p['b1'] for p in layers]).reshape(NUM_LAYERS, 1, FFN)
    b2 = jnp.stack([p['b2'] for p in layers]).reshape(NUM_LAYERS, 1, DIM)
    l1s = jnp.stack([p['ln1_s'] for p in layers]).reshape(NUM_LAYERS, 1, DIM)
    l1b = jnp.stack([p['ln1_b'] for p in layers]).reshape(NUM_LAYERS, 1, DIM)
    l2s = jnp.stack([p['ln2_s'] for p in layers]).reshape(NUM_LAYERS, 1, DIM)
    l2b = jnp.stack([p['ln2_b'] for p in layers]).reshape(NUM_LAYERS, 1, DIM)

    scope_sorted, hole_reprs = pl.pallas_call(
        _encoder_body,
        grid=(NSTEP, NUM_LAYERS),
        in_specs=[
            pl.BlockSpec((1, SEQ, DIM), lambda g, l: (g, 0, 0)),
            pl.BlockSpec((1, SEQ, N_TREES), lambda g, l: (g, 0, 0)),
            pl.BlockSpec((1, 1, SEQ), lambda g, l: (g, 0, 0)),
            _full((SEQ, DIM)),
            _full((SEQ, DIM)),
            _per_layer((DIM, DIM)),
            _per_layer((DIM, DIM)),
            _per_layer((DIM, DIM)),
            _per_layer((DIM, DIM)),
            _per_layer((DIM, FFN)),
            _per_layer((FFN, DIM)),
            _per_layer((1, FFN)),
            _per_layer((1, DIM)),
            _per_layer((1, DIM)),
            _per_layer((1, DIM)),
            _per_layer((1, DIM)),
            _per_layer((1, DIM)),
        ],
        out_specs=[
            _full((N_TREES, DIM)),
            _full((B_HOLE, DIM)),
        ],
        out_shape=[
            jax.ShapeDtypeStruct((N_TREES, DIM), jnp.float32),
            jax.ShapeDtypeStruct((B_HOLE, DIM), jnp.float32),
        ],
        scratch_shapes=[pltpu.VMEM((SEQ, DIM), jnp.float32)],
        compiler_params=pltpu.CompilerParams(
            dimension_semantics=("arbitrary", "arbitrary")),
    )(feat_all, onehot, pbias, cosf, sinf, wq, wk, wv, wo, w1, w2,
      b1, b2, l1s, l1b, l2s, l2b)

    scope_reprs = jnp.take(scope_sorted, inv_order, axis=0)
    return scope_reprs, hole_reprs
